# baseline (speedup 1.0000x reference)
import functools

import jax
import jax.numpy as jnp
from jax import lax
from jax.experimental import pallas as pl
from jax.experimental.pallas import tpu as pltpu
from jax.experimental.pallas import tpu_sc as plsc

N = 10000
NP = 10112
E = 320000
D = 128

NC = 2
NS = 16
CH = 64
NBUF = 5
NCH = 320
NGRP = 10
G = NCH // NGRP
LOOKAHEAD = 3
EPT = NCH * CH
E_PAD = NS * EPT
NH = 10240
COLS = NH // NS
RPT = NP // NS

@functools.cache
def _get_mesh():
    return plsc.VectorSubcoreMesh(
        core_axis_name="c", subcore_axis_name="s", num_cores=NC,
        num_subcores=NS,
    )


@functools.cache
def _degree_kernel_fn():
    return functools.partial(
        pl.kernel,
        out_type=jax.ShapeDtypeStruct((NC * NH,), jnp.float32),
        mesh=_get_mesh(),
        scratch_types=[
            pltpu.VMEM((EPT,), jnp.int32),
            pltpu.VMEM((NH,), jnp.float32),
            pltpu.VMEM((COLS,), jnp.float32),
            pltpu.VMEM((COLS,), jnp.float32),
            pltpu.VMEM_SHARED((NS, NH), jnp.float32),
        ],
        compiler_params=pltpu.CompilerParams(needs_layout_passes=False),
    )(_degree_body)


def _degree_body(dst_hbm, deg_hbm, dst_v, hist_v, racc_v, rtmp_v, shared):
    c = lax.axis_index("c")
    s = lax.axis_index("s")
    pltpu.sync_copy(dst_hbm.at[c, s], dst_v)

    zero16 = jnp.zeros((16,), jnp.float32)
    one16 = jnp.ones((16,), jnp.float32)

    def zbody(i, carry):
        hist_v[pl.ds(i * 16, 16)] = zero16
        return carry

    lax.fori_loop(0, NH // 16, zbody, 0, unroll=4)

    def abody(i, carry):
        idx = dst_v[pl.ds(i * 16, 16)]
        plsc.addupdate_scatter(hist_v, [idx], one16)
        return carry

    lax.fori_loop(0, EPT // 16, abody, 0, unroll=4)

    pltpu.sync_copy(hist_v, shared.at[s])
    plsc.subcore_barrier()

    base = s * COLS
    pltpu.sync_copy(shared.at[0, pl.ds(base, COLS)], racc_v)

    def rbody(k, carry):
        pltpu.sync_copy(shared.at[k, pl.ds(base, COLS)], rtmp_v)

        def addv(j, c2):
            racc_v[pl.ds(j * 16, 16)] = (
                racc_v[pl.ds(j * 16, 16)] + rtmp_v[pl.ds(j * 16, 16)]
            )
            return c2

        lax.fori_loop(0, COLS // 16, addv, 0, unroll=4)
        return carry

    lax.fori_loop(1, NS, rbody, 0)
    pltpu.sync_copy(racc_v, deg_hbm.at[pl.ds(c * NH + base, COLS)])


@functools.cache
def _aggregate_kernel_fn():
    return functools.partial(
        pl.kernel,
        out_type=jax.ShapeDtypeStruct((NC * NP, D), jnp.float32),
        mesh=_get_mesh(),
        scratch_types=[
            pltpu.VMEM((G, CH), jnp.int32),
            pltpu.VMEM((G, CH), jnp.int32),
            [pltpu.VMEM((CH, D), jnp.float32) for _ in range(NBUF)],
            [pltpu.SemaphoreType.DMA for _ in range(NBUF)],
            [pltpu.SemaphoreType.DMA for _ in range(NBUF)],
            pltpu.VMEM_SHARED((NP, D), jnp.float32),
        ],
    )(_aggregate_body)


def _aggregate_body(y_hbm, src_hbm, dst_hbm, out_hbm, src_v, dst_v, bufs,
                    gsems, ssems, acc):
    c = lax.axis_index("c")
    s = lax.axis_index("s")

    pltpu.sync_copy(
        y_hbm.at[pl.ds(c * NP + s * RPT, RPT)], acc.at[pl.ds(s * RPT, RPT)]
    )
    plsc.subcore_barrier()

    def start_gather(ch, b):
        pltpu.async_copy(y_hbm.at[src_v.at[ch]], bufs[b], gsems[b])

    def wait_gather(b):
        pltpu.make_async_copy(y_hbm.at[pl.ds(0, CH)], bufs[b], gsems[b]).wait()

    def start_scatter(ch, b):
        pltpu.async_copy(bufs[b], acc.at[dst_v.at[ch]], ssems[b], add=True)

    def wait_scatter(b):
        pltpu.make_async_copy(bufs[b], acc.at[pl.ds(0, CH)], ssems[b]).wait()

    def grp_body(grp, carry):
        pltpu.sync_copy(src_hbm.at[c, s, pl.ds(grp * G, G)], src_v)
        pltpu.sync_copy(dst_hbm.at[c, s, pl.ds(grp * G, G)], dst_v)

        for b in range(LOOKAHEAD):
            start_gather(b, b)
        for ch in range(G):
            b = ch % NBUF
            wait_gather(b)
            start_scatter(ch, b)
            nch = ch + LOOKAHEAD
            if nch < G:
                bn = nch % NBUF
                if nch >= NBUF:
                    wait_scatter(bn)
                start_gather(nch, bn)
        for b in range(NBUF):
            wait_scatter(b)
        return carry

    lax.fori_loop(0, NGRP, grp_body, 0)

    plsc.subcore_barrier()
    pltpu.sync_copy(
        acc.at[pl.ds(s * RPT, RPT)], out_hbm.at[pl.ds(c * NP + s * RPT, RPT)]
    )


_R = 1264


def _first_body(x_ref, w_ref, dinv_ref, y_ref):
    y_ref[0] = dinv_ref[0] * jnp.dot(
        x_ref[...], w_ref[0], preferred_element_type=jnp.float32
    )


def _mid_body(s_ref, w_ref, b_ref, dinv_ref, y_ref):
    t = jnp.maximum(dinv_ref[0] * s_ref[0] + b_ref[0], 0.0)
    y_ref[0] = dinv_ref[0] * jnp.dot(
        t, w_ref[0], preferred_element_type=jnp.float32
    )


def _final_body(s_ref, dinv_ref, bb_ref, o_ref):
    o_ref[...] = (
        dinv_ref[0] * s_ref[0] + dinv_ref[1] * s_ref[1] + bb_ref[...]
    )


def _tc_first(x, w_all, dinvb):
    return pl.pallas_call(
        _first_body,
        grid=(NC, NP // _R),
        in_specs=[
            pl.BlockSpec((_R, D), lambda k, i: (i, 0)),
            pl.BlockSpec((1, D, D), lambda k, i: (k, 0, 0)),
            pl.BlockSpec((1, _R, D), lambda k, i: (k, i, 0)),
        ],
        out_specs=pl.BlockSpec((1, _R, D), lambda k, i: (k, i, 0)),
        out_shape=jax.ShapeDtypeStruct((NC, NP, D), jnp.float32),
    )(x, w_all, dinvb)


def _tc_mid(s_all, w_all, b_all, dinvb):
    return pl.pallas_call(
        _mid_body,
        grid=(NC, NP // _R),
        in_specs=[
            pl.BlockSpec((1, _R, D), lambda k, i: (k, i, 0)),
            pl.BlockSpec((1, D, D), lambda k, i: (k, 0, 0)),
            pl.BlockSpec((1, 1, D), lambda k, i: (k, 0, 0)),
            pl.BlockSpec((1, _R, D), lambda k, i: (k, i, 0)),
        ],
        out_specs=pl.BlockSpec((1, _R, D), lambda k, i: (k, i, 0)),
        out_shape=jax.ShapeDtypeStruct((NC, NP, D), jnp.float32),
    )(s_all, w_all, b_all, dinvb)


def _tc_final(s_all, dinvb, bb):
    return pl.pallas_call(
        _final_body,
        grid=(NP // _R,),
        in_specs=[
            pl.BlockSpec((NC, _R, D), lambda i: (0, i, 0)),
            pl.BlockSpec((NC, _R, D), lambda i: (0, i, 0)),
            pl.BlockSpec((1, D), lambda i: (0, 0)),
        ],
        out_specs=pl.BlockSpec((_R, D), lambda i: (i, 0)),
        out_shape=jax.ShapeDtypeStruct((NP, D), jnp.float32),
    )(s_all, dinvb, bb)


def kernel(node_feature, edge_index, edge_index_new, W0, b0, W1, b1, W2, b2,
           W3, b3):
    x = node_feature.astype(jnp.float32)
    src1 = edge_index[0].astype(jnp.int32)
    dst1 = edge_index[1].astype(jnp.int32)
    src2 = edge_index_new[0].astype(jnp.int32)
    dst2 = edge_index_new[1].astype(jnp.int32)

    def pad_to(a, fill):
        return jnp.concatenate(
            [a, jnp.full((E_PAD - E,), fill, jnp.int32)]
        ).reshape(NS, NCH, CH)

    o1 = jnp.argsort(src1); src1, dst1 = src1[o1], dst1[o1]
    o2 = jnp.argsort(src2); src2, dst2 = src2[o2], dst2[o2]
    srcs = jnp.stack([pad_to(src1, 0), pad_to(src2 + NP, 0)])
    dsts = jnp.stack([pad_to(dst1, N), pad_to(dst2, N)])
    dsts_flat = dsts.reshape(NC, NS, EPT)
    xp = jnp.pad(x, ((0, NP - N), (0, 0)))

    deg = _degree_kernel_fn()(dsts_flat).reshape(NC, NH)
    dinv = lax.rsqrt(deg[:, :NP] + 1.0)
    dinvb = jnp.broadcast_to(dinv[:, :, None], (NC, NP, D))

    w_first = jnp.stack([W0, W1])
    w_mid = jnp.stack([W2, W3])
    b_first = jnp.stack([b0, b1]).reshape(NC, 1, D)
    bb = (b2 + b3).reshape(1, D)

    y1 = _tc_first(xp, w_first, dinvb).reshape(NC * NP, D)
    s1 = _aggregate_kernel_fn()(y1, srcs, dsts).reshape(NC, NP, D)

    y2 = _tc_mid(s1, w_mid, b_first, dinvb).reshape(NC * NP, D)
    s2 = _aggregate_kernel_fn()(y2, srcs, dsts).reshape(NC, NP, D)

    return _tc_final(s2, dinvb, bb)[:N]


# speedup vs baseline: 1.7808x; 1.7808x over previous
import functools

import jax
import jax.numpy as jnp
from jax import lax
from jax.experimental import pallas as pl
from jax.experimental.pallas import tpu as pltpu
from jax.experimental.pallas import tpu_sc as plsc

N = 10000
NP = 10112
E = 320000
D = 128

NC = 2
NS = 16
CH = 128
DPK = 64
NBUF = 2
NCH = 160
NGRP = 10
G = NCH // NGRP
EPT = NCH * CH
E_PAD = NS * EPT
NH = 10240
COLS = NH // NS
RPT = NP // NS

@functools.cache
def _get_mesh():
    return plsc.VectorSubcoreMesh(
        core_axis_name="c", subcore_axis_name="s", num_cores=NC,
        num_subcores=NS,
    )


@functools.cache
def _degree_kernel_fn():
    return functools.partial(
        pl.kernel,
        out_type=jax.ShapeDtypeStruct((NC * NH,), jnp.float32),
        mesh=_get_mesh(),
        scratch_types=[
            pltpu.VMEM((EPT,), jnp.int32),
            pltpu.VMEM((NH,), jnp.float32),
            pltpu.VMEM((COLS,), jnp.float32),
            pltpu.VMEM((COLS,), jnp.float32),
            pltpu.VMEM_SHARED((NS, NH), jnp.float32),
        ],
        compiler_params=pltpu.CompilerParams(needs_layout_passes=False),
    )(_degree_body)


def _degree_body(dst_hbm, deg_hbm, dst_v, hist_v, racc_v, rtmp_v, shared):
    c = lax.axis_index("c")
    s = lax.axis_index("s")
    pltpu.sync_copy(dst_hbm.at[c, s], dst_v)

    zero16 = jnp.zeros((16,), jnp.float32)
    one16 = jnp.ones((16,), jnp.float32)

    def zbody(i, carry):
        hist_v[pl.ds(i * 16, 16)] = zero16
        return carry

    lax.fori_loop(0, NH // 16, zbody, 0, unroll=4)

    def abody(i, carry):
        idx = dst_v[pl.ds(i * 16, 16)]
        plsc.addupdate_scatter(hist_v, [idx], one16)
        return carry

    lax.fori_loop(0, EPT // 16, abody, 0, unroll=4)

    pltpu.sync_copy(hist_v, shared.at[s])
    plsc.subcore_barrier()

    base = s * COLS
    pltpu.sync_copy(shared.at[0, pl.ds(base, COLS)], racc_v)

    def rbody(k, carry):
        pltpu.sync_copy(shared.at[k, pl.ds(base, COLS)], rtmp_v)

        def addv(j, c2):
            racc_v[pl.ds(j * 16, 16)] = (
                racc_v[pl.ds(j * 16, 16)] + rtmp_v[pl.ds(j * 16, 16)]
            )
            return c2

        lax.fori_loop(0, COLS // 16, addv, 0, unroll=4)
        return carry

    lax.fori_loop(1, NS, rbody, 0)
    pltpu.sync_copy(racc_v, deg_hbm.at[pl.ds(c * NH + base, COLS)])


@functools.cache
def _aggregate_kernel_fn():
    return functools.partial(
        pl.kernel,
        out_type=jax.ShapeDtypeStruct((NC * NP, D), jnp.float32),
        mesh=_get_mesh(),
        scratch_types=[
            pltpu.VMEM((G, CH), jnp.int32),
            pltpu.VMEM((G, CH), jnp.int32),
            [pltpu.VMEM((CH, DPK), jnp.int32) for _ in range(NBUF)],
            pltpu.VMEM((CH, D), jnp.float32),
            [pltpu.SemaphoreType.DMA for _ in range(NBUF)],
            pltpu.SemaphoreType.DMA,
            pltpu.VMEM_SHARED((NP, D), jnp.float32),
        ],
        compiler_params=pltpu.CompilerParams(
            needs_layout_passes=False, use_tc_tiling_on_sc=False
        ),
    )(_aggregate_body)


def _aggregate_body(y_hbm, ypk_hbm, src_hbm, dst_hbm, out_hbm, src_v, dst_v,
                    pbufs, fbuf, gsems, ssem, acc):
    c = lax.axis_index("c")
    s = lax.axis_index("s")

    pltpu.sync_copy(
        y_hbm.at[pl.ds(c * NP + s * RPT, RPT)], acc.at[pl.ds(s * RPT, RPT)]
    )
    plsc.subcore_barrier()

    def start_gather(ch, b):
        pltpu.async_copy(ypk_hbm.at[src_v.at[ch]], pbufs[b], gsems[b])

    def wait_gather(b):
        pltpu.make_async_copy(
            ypk_hbm.at[pl.ds(0, CH)], pbufs[b], gsems[b]
        ).wait()

    def start_scatter(ch):
        pltpu.async_copy(fbuf, acc.at[dst_v.at[ch]], ssem, add=True)

    def wait_scatter():
        pltpu.make_async_copy(fbuf, acc.at[pl.ds(0, CH)], ssem).wait()

    hmask = jnp.full((16,), -65536, jnp.int32)

    def unpack(b):
        def urow(r, carry):
            for q in range(DPK // 16):
                v = pbufs[b][r, pl.ds(q * 16, 16)]
                flo = plsc.bitcast(lax.shift_left(v, 16), jnp.float32)
                fhi = plsc.bitcast(lax.bitwise_and(v, hmask), jnp.float32)
                fbuf[r, pl.ds(q * 32, 16)] = flo
                fbuf[r, pl.ds(q * 32 + 16, 16)] = fhi
            return carry

        lax.fori_loop(0, CH, urow, 0)

    def grp_body(grp, carry):
        pltpu.sync_copy(src_hbm.at[c, s, pl.ds(grp * G, G)], src_v)
        pltpu.sync_copy(dst_hbm.at[c, s, pl.ds(grp * G, G)], dst_v)

        for b in range(NBUF):
            start_gather(b, b)
        for ch in range(G):
            b = ch % NBUF
            wait_gather(b)
            if ch > 0:
                wait_scatter()
            unpack(b)
            nch = ch + NBUF
            if nch < G:
                start_gather(nch, b)
            start_scatter(ch)
        wait_scatter()
        return carry

    lax.fori_loop(0, NGRP, grp_body, 0)

    plsc.subcore_barrier()
    pltpu.sync_copy(
        acc.at[pl.ds(s * RPT, RPT)], out_hbm.at[pl.ds(c * NP + s * RPT, RPT)]
    )


_R = 1264


def _first_body(x_ref, w_ref, dinv_ref, y_ref):
    y_ref[0] = dinv_ref[0] * jnp.dot(
        x_ref[...], w_ref[0], preferred_element_type=jnp.float32
    )


def _mid_body(s_ref, w_ref, b_ref, dinv_ref, y_ref):
    t = jnp.maximum(dinv_ref[0] * s_ref[0] + b_ref[0], 0.0)
    y_ref[0] = dinv_ref[0] * jnp.dot(
        t, w_ref[0], preferred_element_type=jnp.float32
    )


def _final_body(s_ref, dinv_ref, bb_ref, o_ref):
    o_ref[...] = (
        dinv_ref[0] * s_ref[0] + dinv_ref[1] * s_ref[1] + bb_ref[...]
    )


def _tc_first(x, w_all, dinvb):
    return pl.pallas_call(
        _first_body,
        grid=(NC, NP // _R),
        in_specs=[
            pl.BlockSpec((_R, D), lambda k, i: (i, 0)),
            pl.BlockSpec((1, D, D), lambda k, i: (k, 0, 0)),
            pl.BlockSpec((1, _R, D), lambda k, i: (k, i, 0)),
        ],
        out_specs=pl.BlockSpec((1, _R, D), lambda k, i: (k, i, 0)),
        out_shape=jax.ShapeDtypeStruct((NC, NP, D), jnp.float32),
    )(x, w_all, dinvb)


def _tc_mid(s_all, w_all, b_all, dinvb):
    return pl.pallas_call(
        _mid_body,
        grid=(NC, NP // _R),
        in_specs=[
            pl.BlockSpec((1, _R, D), lambda k, i: (k, i, 0)),
            pl.BlockSpec((1, D, D), lambda k, i: (k, 0, 0)),
            pl.BlockSpec((1, 1, D), lambda k, i: (k, 0, 0)),
            pl.BlockSpec((1, _R, D), lambda k, i: (k, i, 0)),
        ],
        out_specs=pl.BlockSpec((1, _R, D), lambda k, i: (k, i, 0)),
        out_shape=jax.ShapeDtypeStruct((NC, NP, D), jnp.float32),
    )(s_all, w_all, b_all, dinvb)


def _tc_final(s_all, dinvb, bb):
    return pl.pallas_call(
        _final_body,
        grid=(NP // _R,),
        in_specs=[
            pl.BlockSpec((NC, _R, D), lambda i: (0, i, 0)),
            pl.BlockSpec((NC, _R, D), lambda i: (0, i, 0)),
            pl.BlockSpec((1, D), lambda i: (0, 0)),
        ],
        out_specs=pl.BlockSpec((_R, D), lambda i: (i, 0)),
        out_shape=jax.ShapeDtypeStruct((NP, D), jnp.float32),
    )(s_all, dinvb, bb)


def _pack_rows(y):
    yb = lax.bitcast_convert_type(
        y.astype(jnp.bfloat16).reshape(NC * NP, 4, 2, 16), jnp.uint16
    )
    w = yb[:, :, 0, :].astype(jnp.uint32) | (
        yb[:, :, 1, :].astype(jnp.uint32) << 16
    )
    return lax.bitcast_convert_type(w, jnp.int32).reshape(NC * NP, DPK)


def kernel(node_feature, edge_index, edge_index_new, W0, b0, W1, b1, W2, b2,
           W3, b3):
    x = node_feature.astype(jnp.float32)
    src1 = edge_index[0].astype(jnp.int32)
    dst1 = edge_index[1].astype(jnp.int32)
    src2 = edge_index_new[0].astype(jnp.int32)
    dst2 = edge_index_new[1].astype(jnp.int32)

    def pad_to(a, fill):
        return jnp.concatenate(
            [a, jnp.full((E_PAD - E,), fill, jnp.int32)]
        ).reshape(NS, NCH, CH)

    srcs = jnp.stack([pad_to(src1, 0), pad_to(src2 + NP, 0)])
    dsts = jnp.stack([pad_to(dst1, N), pad_to(dst2, N)])
    dsts_flat = dsts.reshape(NC, NS, EPT)
    xp = jnp.pad(x, ((0, NP - N), (0, 0)))

    deg = _degree_kernel_fn()(dsts_flat).reshape(NC, NH)
    dinv = lax.rsqrt(deg[:, :NP] + 1.0)
    dinvb = jnp.broadcast_to(dinv[:, :, None], (NC, NP, D))

    w_first = jnp.stack([W0, W1])
    w_mid = jnp.stack([W2, W3])
    b_first = jnp.stack([b0, b1]).reshape(NC, 1, D)
    bb = (b2 + b3).reshape(1, D)

    y1 = _tc_first(xp, w_first, dinvb).reshape(NC * NP, D)
    s1 = _aggregate_kernel_fn()(y1, _pack_rows(y1), srcs, dsts)
    s1 = s1.reshape(NC, NP, D)

    y2 = _tc_mid(s1, w_mid, b_first, dinvb).reshape(NC * NP, D)
    s2 = _aggregate_kernel_fn()(y2, _pack_rows(y2), srcs, dsts)
    s2 = s2.reshape(NC, NP, D)

    return _tc_final(s2, dinvb, bb)[:N]


# speedup vs baseline: 1.8848x; 1.0584x over previous
import functools

import jax
import jax.numpy as jnp
from jax import lax
from jax.experimental import pallas as pl
from jax.experimental.pallas import tpu as pltpu
from jax.experimental.pallas import tpu_sc as plsc

N = 10000
NP = 10112
E = 320000
D = 128

NC = 2
NS = 16
CH = 128
DPK = 64
NBUF = 2
NCH = 160
NGRP = 10
G = NCH // NGRP
EPT = NCH * CH
E_PAD = NS * EPT
NH = 10240
COLS = NH // NS
RPT = NP // NS

@functools.cache
def _get_mesh():
    return plsc.VectorSubcoreMesh(
        core_axis_name="c", subcore_axis_name="s", num_cores=NC,
        num_subcores=NS,
    )


@functools.cache
def _degree_kernel_fn():
    return functools.partial(
        pl.kernel,
        out_type=jax.ShapeDtypeStruct((NC * NH,), jnp.float32),
        mesh=_get_mesh(),
        scratch_types=[
            pltpu.VMEM((EPT,), jnp.int32),
            pltpu.VMEM((NH,), jnp.float32),
            pltpu.VMEM((COLS,), jnp.float32),
            pltpu.VMEM((COLS,), jnp.float32),
            pltpu.VMEM_SHARED((NS, NH), jnp.float32),
        ],
        compiler_params=pltpu.CompilerParams(needs_layout_passes=False),
    )(_degree_body)


def _degree_body(dst_hbm, deg_hbm, dst_v, hist_v, racc_v, rtmp_v, shared):
    c = lax.axis_index("c")
    s = lax.axis_index("s")
    pltpu.sync_copy(dst_hbm.at[c, s], dst_v)

    zero16 = jnp.zeros((16,), jnp.float32)
    one16 = jnp.ones((16,), jnp.float32)

    def zbody(i, carry):
        hist_v[pl.ds(i * 16, 16)] = zero16
        return carry

    lax.fori_loop(0, NH // 16, zbody, 0, unroll=4)

    def abody(i, carry):
        idx = dst_v[pl.ds(i * 16, 16)]
        plsc.addupdate_scatter(hist_v, [idx], one16)
        return carry

    lax.fori_loop(0, EPT // 16, abody, 0, unroll=4)

    pltpu.sync_copy(hist_v, shared.at[s])
    plsc.subcore_barrier()

    base = s * COLS
    pltpu.sync_copy(shared.at[0, pl.ds(base, COLS)], racc_v)

    def rbody(k, carry):
        pltpu.sync_copy(shared.at[k, pl.ds(base, COLS)], rtmp_v)

        def addv(j, c2):
            racc_v[pl.ds(j * 16, 16)] = (
                racc_v[pl.ds(j * 16, 16)] + rtmp_v[pl.ds(j * 16, 16)]
            )
            return c2

        lax.fori_loop(0, COLS // 16, addv, 0, unroll=4)
        return carry

    lax.fori_loop(1, NS, rbody, 0)
    pltpu.sync_copy(racc_v, deg_hbm.at[pl.ds(c * NH + base, COLS)])


@functools.cache
def _aggregate_kernel_fn():
    return functools.partial(
        pl.kernel,
        out_type=jax.ShapeDtypeStruct((NC * NP, D), jnp.float32),
        mesh=_get_mesh(),
        scratch_types=[
            pltpu.VMEM((G, CH), jnp.int32),
            pltpu.VMEM((G, CH), jnp.int32),
            [pltpu.VMEM((CH, DPK), jnp.int32) for _ in range(NBUF)],
            pltpu.VMEM((CH, D), jnp.float32),
            [pltpu.SemaphoreType.DMA for _ in range(NBUF)],
            pltpu.SemaphoreType.DMA,
            pltpu.VMEM_SHARED((NP, D), jnp.float32),
        ],
        compiler_params=pltpu.CompilerParams(
            needs_layout_passes=False, use_tc_tiling_on_sc=False
        ),
    )(_aggregate_body)


def _aggregate_body(y_hbm, ypk_hbm, src_hbm, dst_hbm, out_hbm, src_v, dst_v,
                    pbufs, fbuf, gsems, ssem, acc):
    c = lax.axis_index("c")
    s = lax.axis_index("s")

    pltpu.sync_copy(
        y_hbm.at[pl.ds(c * NP + s * RPT, RPT)], acc.at[pl.ds(s * RPT, RPT)]
    )
    plsc.subcore_barrier()

    def start_gather(ch, b):
        pltpu.async_copy(ypk_hbm.at[src_v.at[ch]], pbufs[b], gsems[b])

    def wait_gather(b):
        pltpu.make_async_copy(
            ypk_hbm.at[pl.ds(0, CH)], pbufs[b], gsems[b]
        ).wait()

    def start_scatter(ch):
        pltpu.async_copy(fbuf, acc.at[dst_v.at[ch]], ssem, add=True)

    def wait_scatter():
        pltpu.make_async_copy(fbuf, acc.at[pl.ds(0, CH)], ssem).wait()

    hmask = jnp.full((16,), -65536, jnp.int32)

    def unpack(b):
        def urow(r, carry):
            for q in range(DPK // 16):
                v = pbufs[b][r, pl.ds(q * 16, 16)]
                flo = plsc.bitcast(lax.shift_left(v, 16), jnp.float32)
                fhi = plsc.bitcast(lax.bitwise_and(v, hmask), jnp.float32)
                fbuf[r, pl.ds(q * 32, 16)] = flo
                fbuf[r, pl.ds(q * 32 + 16, 16)] = fhi
            return carry

        lax.fori_loop(0, CH, urow, 0)

    def grp_body(grp, carry):
        pltpu.sync_copy(src_hbm.at[c, s, pl.ds(grp * G, G)], src_v)
        pltpu.sync_copy(dst_hbm.at[c, s, pl.ds(grp * G, G)], dst_v)

        for b in range(NBUF):
            start_gather(b, b)
        for ch in range(G):
            b = ch % NBUF
            wait_gather(b)
            if ch > 0:
                wait_scatter()
            unpack(b)
            nch = ch + NBUF
            if nch < G:
                start_gather(nch, b)
            start_scatter(ch)
        wait_scatter()
        return carry

    lax.fori_loop(0, NGRP, grp_body, 0)

    plsc.subcore_barrier()
    pltpu.sync_copy(
        acc.at[pl.ds(s * RPT, RPT)], out_hbm.at[pl.ds(c * NP + s * RPT, RPT)]
    )


_R = 1264


def _pack_block(y):
    yb = lax.bitcast_convert_type(y.astype(jnp.bfloat16), jnp.uint16)
    lo = jnp.concatenate([yb[..., 32 * q:32 * q + 16] for q in range(4)], -1)
    hi = jnp.concatenate(
        [yb[..., 32 * q + 16:32 * q + 32] for q in range(4)], -1
    )
    w = lo.astype(jnp.uint32) | (hi.astype(jnp.uint32) << 16)
    return lax.bitcast_convert_type(w, jnp.int32)


def _first_body(x_ref, w_ref, dinv_ref, y_ref, ypk_ref):
    y = dinv_ref[0] * jnp.dot(
        x_ref[...], w_ref[0], preferred_element_type=jnp.float32
    )
    y_ref[0] = y
    ypk_ref[0] = _pack_block(y)


def _mid_body(s_ref, w_ref, b_ref, dinv_ref, y_ref, ypk_ref):
    t = jnp.maximum(dinv_ref[0] * s_ref[0] + b_ref[0], 0.0)
    y = dinv_ref[0] * jnp.dot(
        t, w_ref[0], preferred_element_type=jnp.float32
    )
    y_ref[0] = y
    ypk_ref[0] = _pack_block(y)


def _final_body(s_ref, dinv_ref, bb_ref, o_ref):
    o_ref[...] = (
        dinv_ref[0] * s_ref[0] + dinv_ref[1] * s_ref[1] + bb_ref[...]
    )


def _tc_first(x, w_all, dinvb):
    return pl.pallas_call(
        _first_body,
        grid=(NC, NP // _R),
        in_specs=[
            pl.BlockSpec((_R, D), lambda k, i: (i, 0)),
            pl.BlockSpec((1, D, D), lambda k, i: (k, 0, 0)),
            pl.BlockSpec((1, _R, D), lambda k, i: (k, i, 0)),
        ],
        out_specs=[
            pl.BlockSpec((1, _R, D), lambda k, i: (k, i, 0)),
            pl.BlockSpec((1, _R, DPK), lambda k, i: (k, i, 0)),
        ],
        out_shape=[
            jax.ShapeDtypeStruct((NC, NP, D), jnp.float32),
            jax.ShapeDtypeStruct((NC, NP, DPK), jnp.int32),
        ],
    )(x, w_all, dinvb)


def _tc_mid(s_all, w_all, b_all, dinvb):
    return pl.pallas_call(
        _mid_body,
        grid=(NC, NP // _R),
        in_specs=[
            pl.BlockSpec((1, _R, D), lambda k, i: (k, i, 0)),
            pl.BlockSpec((1, D, D), lambda k, i: (k, 0, 0)),
            pl.BlockSpec((1, 1, D), lambda k, i: (k, 0, 0)),
            pl.BlockSpec((1, _R, D), lambda k, i: (k, i, 0)),
        ],
        out_specs=[
            pl.BlockSpec((1, _R, D), lambda k, i: (k, i, 0)),
            pl.BlockSpec((1, _R, DPK), lambda k, i: (k, i, 0)),
        ],
        out_shape=[
            jax.ShapeDtypeStruct((NC, NP, D), jnp.float32),
            jax.ShapeDtypeStruct((NC, NP, DPK), jnp.int32),
        ],
    )(s_all, w_all, b_all, dinvb)


def _tc_final(s_all, dinvb, bb):
    return pl.pallas_call(
        _final_body,
        grid=(NP // _R,),
        in_specs=[
            pl.BlockSpec((NC, _R, D), lambda i: (0, i, 0)),
            pl.BlockSpec((NC, _R, D), lambda i: (0, i, 0)),
            pl.BlockSpec((1, D), lambda i: (0, 0)),
        ],
        out_specs=pl.BlockSpec((_R, D), lambda i: (i, 0)),
        out_shape=jax.ShapeDtypeStruct((NP, D), jnp.float32),
    )(s_all, dinvb, bb)


def kernel(node_feature, edge_index, edge_index_new, W0, b0, W1, b1, W2, b2,
           W3, b3):
    x = node_feature.astype(jnp.float32)
    src1 = edge_index[0].astype(jnp.int32)
    dst1 = edge_index[1].astype(jnp.int32)
    src2 = edge_index_new[0].astype(jnp.int32)
    dst2 = edge_index_new[1].astype(jnp.int32)

    def pad_to(a, fill):
        return jnp.concatenate(
            [a, jnp.full((E_PAD - E,), fill, jnp.int32)]
        ).reshape(NS, NCH, CH)

    srcs = jnp.stack([pad_to(src1, 0), pad_to(src2 + NP, 0)])
    dsts = jnp.stack([pad_to(dst1, N), pad_to(dst2, N)])
    dsts_flat = dsts.reshape(NC, NS, EPT)
    xp = jnp.pad(x, ((0, NP - N), (0, 0)))

    deg = _degree_kernel_fn()(dsts_flat).reshape(NC, NH)
    dinv = lax.rsqrt(deg[:, :NP] + 1.0)
    dinvb = jnp.broadcast_to(dinv[:, :, None], (NC, NP, D))

    w_first = jnp.stack([W0, W1])
    w_mid = jnp.stack([W2, W3])
    b_first = jnp.stack([b0, b1]).reshape(NC, 1, D)
    bb = (b2 + b3).reshape(1, D)

    y1, y1pk = _tc_first(xp, w_first, dinvb)
    s1 = _aggregate_kernel_fn()(
        y1.reshape(NC * NP, D), y1pk.reshape(NC * NP, DPK), srcs, dsts
    ).reshape(NC, NP, D)

    y2, y2pk = _tc_mid(s1, w_mid, b_first, dinvb)
    s2 = _aggregate_kernel_fn()(
        y2.reshape(NC * NP, D), y2pk.reshape(NC * NP, DPK), srcs, dsts
    ).reshape(NC, NP, D)

    return _tc_final(s2, dinvb, bb)[:N]
